# baseline (speedup 1.0000x reference)
import functools

import jax
import jax.numpy as jnp
from jax import lax
from jax.experimental import pallas as pl
from jax.experimental.pallas import tpu as pltpu
from jax.experimental.pallas import tpu_sc as plsc

_N = 16384
_L = 4096
_NW = 32
_RPW = _N // _NW
_CH = 8
_NCH = _RPW // _CH


def _sc_body(x_hbm, y_hbm, out_hbm, buf0, buf1, buf2, xbuf, ybuf,
             sem0, sem1, sem2):
    wid = lax.axis_index("s") * 2 + lax.axis_index("c")
    row0 = wid * _RPW

    zeros16 = jnp.zeros((16,), jnp.float32)
    ones16 = jnp.full((16,), 1.0, jnp.float32)
    zidx16 = jnp.zeros((16,), jnp.int32)
    lane = lax.broadcasted_iota(jnp.int32, (16,), 0)
    row_in_chunk = lane & 7
    col_half = lane >> 3

    bufs = (buf0, buf1, buf2)
    sems = (sem0, sem1, sem2)

    def _zero(j, carry):
        for r in range(_CH):
            buf0[r, pl.ds(j * 16, 16)] = zeros16
            buf1[r, pl.ds(j * 16, 16)] = zeros16
            buf2[r, pl.ds(j * 16, 16)] = zeros16
        return carry

    lax.fori_loop(0, _L // 16, _zero, 0)

    pltpu.sync_copy(x_hbm.at[pl.ds(row0, _RPW)], xbuf)
    pltpu.sync_copy(y_hbm.at[pl.ds(row0, _RPW)], ybuf)

    prev_cols = [None, None, None]
    for ch in range(_NCH):
        b = ch % 3
        buf, sem = bufs[b], sems[b]
        if prev_cols[b] is not None:
            pltpu.make_async_copy(buf, out_hbm.at[pl.ds(0, _CH)], sem).wait()
            prev = prev_cols[b]
            plsc.store_scatter(buf, [row_in_chunk, prev], zeros16)
            plsc.store_scatter(buf, [row_in_chunk, prev + 1], zeros16)
        xv = plsc.load_gather(xbuf, [ch * _CH + row_in_chunk])
        yv = plsc.load_gather(ybuf, [ch * _CH + row_in_chunk])
        col = 128 * xv + 2 * yv + 64 * col_half
        plsc.store_scatter(buf, [row_in_chunk, col], ones16)
        plsc.store_scatter(buf, [row_in_chunk, col + 1], ones16)
        prev_cols[b] = col
        pltpu.make_async_copy(
            buf, out_hbm.at[pl.ds(row0 + ch * _CH, _CH)], sem).start()
    for b in range(3):
        pltpu.make_async_copy(
            bufs[b], out_hbm.at[pl.ds(0, _CH)], sems[b]).wait()


def kernel(mask, new_glimpse_x, new_glimpse_y):
    n, l = mask.shape
    x = new_glimpse_x.reshape((n,)).astype(jnp.int32)
    y = new_glimpse_y.reshape((n,)).astype(jnp.int32)
    run = functools.partial(
        pl.kernel,
        out_type=jax.ShapeDtypeStruct((n, l), jnp.float32),
        mesh=plsc.VectorSubcoreMesh(core_axis_name="c", subcore_axis_name="s"),
        compiler_params=pltpu.CompilerParams(needs_layout_passes=False),
        scratch_types=[
            pltpu.VMEM((_CH, _L), jnp.float32),
            pltpu.VMEM((_CH, _L), jnp.float32),
            pltpu.VMEM((_CH, _L), jnp.float32),
            pltpu.VMEM((_RPW,), jnp.int32),
            pltpu.VMEM((_RPW,), jnp.int32),
            pltpu.SemaphoreType.DMA,
            pltpu.SemaphoreType.DMA,
            pltpu.SemaphoreType.DMA,
        ],
    )(_sc_body)
    return run(x, y)


# speedup vs baseline: 1.0476x; 1.0476x over previous
import functools

import jax
import jax.numpy as jnp
from jax import lax
from jax.experimental import pallas as pl
from jax.experimental.pallas import tpu as pltpu
from jax.experimental.pallas import tpu_sc as plsc

_N = 16384
_L = 4096
_NW = 32
_RPW = _N // _NW
_CH = 8
_NCH = _RPW // _CH


def _sc_body(x_hbm, y_hbm, out_hbm, buf0, buf1, xbuf, ybuf,
             sem0, sem1, semx):
    wid = lax.axis_index("s") * 2 + lax.axis_index("c")
    row0 = wid * _RPW

    zeros16 = jnp.zeros((16,), jnp.float32)
    ones16 = jnp.full((16,), 1.0, jnp.float32)
    zidx16 = jnp.zeros((16,), jnp.int32)
    lane = lax.broadcasted_iota(jnp.int32, (16,), 0)
    row_in_chunk = lane & 7
    col_half = lane >> 3

    bufs = (buf0, buf1)
    sems = (sem0, sem1)

    xcopy = pltpu.make_async_copy(x_hbm.at[pl.ds(row0, _RPW)], xbuf, semx)
    ycopy = pltpu.make_async_copy(y_hbm.at[pl.ds(row0, _RPW)], ybuf, semx)
    xcopy.start()
    ycopy.start()

    def _zero(j, carry):
        for r in range(_CH):
            buf0[r, pl.ds(j * 16, 16)] = zeros16
            buf1[r, pl.ds(j * 16, 16)] = zeros16
        return carry

    lax.fori_loop(0, _L // 16, _zero, 0)
    xcopy.wait()
    ycopy.wait()

    def _iter(i, prev):
        new = []
        for b in (0, 1):
            buf, sem = bufs[b], sems[b]
            ch = 2 * i + b
            xv = plsc.load_gather(xbuf, [ch * _CH + row_in_chunk])
            yv = plsc.load_gather(ybuf, [ch * _CH + row_in_chunk])
            col = 128 * xv + 2 * yv + 64 * col_half

            @pl.when(i > 0)
            def _drain():
                pltpu.make_async_copy(
                    buf, out_hbm.at[pl.ds(0, _CH)], sem).wait()
                plsc.store_scatter(buf, [row_in_chunk, prev[b]], zeros16)
                plsc.store_scatter(buf, [row_in_chunk, prev[b] + 1], zeros16)

            plsc.store_scatter(buf, [row_in_chunk, col], ones16)
            plsc.store_scatter(buf, [row_in_chunk, col + 1], ones16)
            pltpu.make_async_copy(
                buf, out_hbm.at[pl.ds(row0 + ch * _CH, _CH)], sem).start()
            new.append(col)
        return tuple(new)

    lax.fori_loop(0, _NCH // 2, _iter, (zidx16, zidx16))
    for b in (0, 1):
        pltpu.make_async_copy(
            bufs[b], out_hbm.at[pl.ds(0, _CH)], sems[b]).wait()


def kernel(mask, new_glimpse_x, new_glimpse_y):
    n, l = mask.shape
    x = new_glimpse_x.reshape((n,)).astype(jnp.int32)
    y = new_glimpse_y.reshape((n,)).astype(jnp.int32)
    run = functools.partial(
        pl.kernel,
        out_type=jax.ShapeDtypeStruct((n, l), jnp.float32),
        mesh=plsc.VectorSubcoreMesh(core_axis_name="c", subcore_axis_name="s"),
        compiler_params=pltpu.CompilerParams(needs_layout_passes=False),
        scratch_types=[
            pltpu.VMEM((_CH, _L), jnp.float32),
            pltpu.VMEM((_CH, _L), jnp.float32),
            pltpu.VMEM((_RPW,), jnp.int32),
            pltpu.VMEM((_RPW,), jnp.int32),
            pltpu.SemaphoreType.DMA,
            pltpu.SemaphoreType.DMA,
            pltpu.SemaphoreType.DMA,
        ],
    )(_sc_body)
    return run(x, y)


# speedup vs baseline: 1.0505x; 1.0029x over previous
import functools

import jax
import jax.numpy as jnp
from jax import lax
from jax.experimental import pallas as pl
from jax.experimental.pallas import tpu as pltpu
from jax.experimental.pallas import tpu_sc as plsc

_N = 16384
_L = 4096
_NW = 32
_RPW = _N // _NW
_CH = 8
_NCH = _RPW // _CH


def _sc_body(x_hbm, y_hbm, out_hbm, buf0, buf1, xbuf, ybuf,
             sem0, sem1, semx):
    wid = lax.axis_index("s") * 2 + lax.axis_index("c")
    row0 = wid * _RPW

    zeros16 = jnp.zeros((16,), jnp.float32)
    ones16 = jnp.full((16,), 1.0, jnp.float32)
    zidx16 = jnp.zeros((16,), jnp.int32)
    lane = lax.broadcasted_iota(jnp.int32, (16,), 0)
    row_in_chunk = lane & 7
    col_half = lane >> 3

    bufs = (buf0, buf1)
    sems = (sem0, sem1)

    xcopy = pltpu.make_async_copy(x_hbm.at[pl.ds(row0, _RPW)], xbuf, semx)
    ycopy = pltpu.make_async_copy(y_hbm.at[pl.ds(row0, _RPW)], ybuf, semx)
    xcopy.start()
    ycopy.start()

    def _zero(buf):
        def _z(j, carry):
            for r in range(_CH):
                buf[r, pl.ds(j * 16, 16)] = zeros16
            return carry
        lax.fori_loop(0, _L // 16, _z, 0)

    def _fill_and_send(ch, buf, sem):
        xv = plsc.load_gather(xbuf, [ch * _CH + row_in_chunk])
        yv = plsc.load_gather(ybuf, [ch * _CH + row_in_chunk])
        col = 128 * xv + 2 * yv + 64 * col_half
        plsc.store_scatter(buf, [row_in_chunk, col], ones16)
        plsc.store_scatter(buf, [row_in_chunk, col + 1], ones16)
        pltpu.make_async_copy(
            buf, out_hbm.at[pl.ds(row0 + ch * _CH, _CH)], sem).start()
        return col

    _zero(buf0)
    xcopy.wait()
    ycopy.wait()
    col0 = _fill_and_send(0, buf0, sem0)
    _zero(buf1)
    col1 = _fill_and_send(1, buf1, sem1)

    def _iter(i, prev):
        new = []
        for b in (0, 1):
            buf, sem = bufs[b], sems[b]
            ch = 2 * i + b
            xv = plsc.load_gather(xbuf, [ch * _CH + row_in_chunk])
            yv = plsc.load_gather(ybuf, [ch * _CH + row_in_chunk])
            col = 128 * xv + 2 * yv + 64 * col_half
            pltpu.make_async_copy(
                buf, out_hbm.at[pl.ds(0, _CH)], sem).wait()
            plsc.store_scatter(buf, [row_in_chunk, prev[b]], zeros16)
            plsc.store_scatter(buf, [row_in_chunk, prev[b] + 1], zeros16)
            plsc.store_scatter(buf, [row_in_chunk, col], ones16)
            plsc.store_scatter(buf, [row_in_chunk, col + 1], ones16)
            pltpu.make_async_copy(
                buf, out_hbm.at[pl.ds(row0 + ch * _CH, _CH)], sem).start()
            new.append(col)
        return tuple(new)

    lax.fori_loop(1, _NCH // 2, _iter, (col0, col1))
    for b in (0, 1):
        pltpu.make_async_copy(
            bufs[b], out_hbm.at[pl.ds(0, _CH)], sems[b]).wait()


def kernel(mask, new_glimpse_x, new_glimpse_y):
    n, l = mask.shape
    x = new_glimpse_x.reshape((n,)).astype(jnp.int32)
    y = new_glimpse_y.reshape((n,)).astype(jnp.int32)
    run = functools.partial(
        pl.kernel,
        out_type=jax.ShapeDtypeStruct((n, l), jnp.float32),
        mesh=plsc.VectorSubcoreMesh(core_axis_name="c", subcore_axis_name="s"),
        compiler_params=pltpu.CompilerParams(needs_layout_passes=False),
        scratch_types=[
            pltpu.VMEM((_CH, _L), jnp.float32),
            pltpu.VMEM((_CH, _L), jnp.float32),
            pltpu.VMEM((_RPW,), jnp.int32),
            pltpu.VMEM((_RPW,), jnp.int32),
            pltpu.SemaphoreType.DMA,
            pltpu.SemaphoreType.DMA,
            pltpu.SemaphoreType.DMA,
        ],
    )(_sc_body)
    return run(x, y)


# speedup vs baseline: 1.0561x; 1.0053x over previous
import functools

import jax
import jax.numpy as jnp
from jax import lax
from jax.experimental import pallas as pl
from jax.experimental.pallas import tpu as pltpu
from jax.experimental.pallas import tpu_sc as plsc

_N = 16384
_L = 4096
_NW = 32
_RPW = _N // _NW
_CH = 8
_NCH = _RPW // _CH


def _sc_body(x_hbm, y_hbm, out_hbm, buf0, buf1, xbuf, ybuf,
             sem0, sem1, semx):
    wid = lax.axis_index("s") * 2 + lax.axis_index("c")
    row0 = wid * _RPW

    zeros16 = jnp.zeros((16,), jnp.float32)
    ones16 = jnp.full((16,), 1.0, jnp.float32)
    lane = lax.broadcasted_iota(jnp.int32, (16,), 0)
    row_in_chunk = lane & 7
    col_half = lane >> 3

    bufs = (buf0, buf1)
    sems = (sem0, sem1)

    xcopy = pltpu.make_async_copy(x_hbm.at[pl.ds(row0, _RPW)], xbuf, semx)
    ycopy = pltpu.make_async_copy(y_hbm.at[pl.ds(row0, _RPW)], ybuf, semx)
    xcopy.start()
    ycopy.start()

    def _zero(buf):
        def _z(j, carry):
            for r in range(_CH):
                buf[r, pl.ds(j * 16, 16)] = zeros16
            return carry
        lax.fori_loop(0, _L // 16, _z, 0)

    def _fill_and_send(ch, buf, sem):
        xv = plsc.load_gather(xbuf, [ch * _CH + row_in_chunk])
        yv = plsc.load_gather(ybuf, [ch * _CH + row_in_chunk])
        col = 128 * xv + 2 * yv + 64 * col_half
        plsc.store_scatter(buf, [row_in_chunk, col], ones16)
        plsc.store_scatter(buf, [row_in_chunk, col + 1], ones16)
        pltpu.make_async_copy(
            buf, out_hbm.at[pl.ds(row0 + ch * _CH, _CH)], sem).start()
        return col

    _zero(buf0)
    xcopy.wait()
    ycopy.wait()
    col0 = _fill_and_send(0, buf0, sem0)
    _zero(buf1)
    col1 = _fill_and_send(1, buf1, sem1)

    def _iter(i, prev):
        new = []
        for b in (0, 1):
            buf, sem = bufs[b], sems[b]
            ch = 2 * i + b
            xv = plsc.load_gather(xbuf, [ch * _CH + row_in_chunk])
            yv = plsc.load_gather(ybuf, [ch * _CH + row_in_chunk])
            col = 128 * xv + 2 * yv + 64 * col_half
            pltpu.make_async_copy(
                buf, out_hbm.at[pl.ds(0, _CH)], sem).wait()
            plsc.store_scatter(buf, [row_in_chunk, prev[b]], zeros16)
            plsc.store_scatter(buf, [row_in_chunk, prev[b] + 1], zeros16)
            plsc.store_scatter(buf, [row_in_chunk, col], ones16)
            plsc.store_scatter(buf, [row_in_chunk, col + 1], ones16)
            pltpu.make_async_copy(
                buf, out_hbm.at[pl.ds(row0 + ch * _CH, _CH)], sem).start()
            new.append(col)
        return tuple(new)

    lax.fori_loop(1, _NCH // 2, _iter, (col0, col1))
    for b in (0, 1):
        pltpu.make_async_copy(
            bufs[b], out_hbm.at[pl.ds(0, _CH)], sems[b]).wait()


def kernel(mask, new_glimpse_x, new_glimpse_y):
    n, l = mask.shape
    x = new_glimpse_x.reshape((n,)).astype(jnp.int32)
    y = new_glimpse_y.reshape((n,)).astype(jnp.int32)
    run = functools.partial(
        pl.kernel,
        out_type=jax.ShapeDtypeStruct((n, l), jnp.float32),
        mesh=plsc.VectorSubcoreMesh(core_axis_name="c", subcore_axis_name="s"),
        compiler_params=pltpu.CompilerParams(needs_layout_passes=False),
        scratch_types=[
            pltpu.VMEM((_CH, _L), jnp.float32),
            pltpu.VMEM((_CH, _L), jnp.float32),
            pltpu.VMEM((_RPW,), jnp.int32),
            pltpu.VMEM((_RPW,), jnp.int32),
            pltpu.SemaphoreType.DMA,
            pltpu.SemaphoreType.DMA,
            pltpu.SemaphoreType.DMA,
        ],
    )(_sc_body)
    return run(x, y)
